# baseline (speedup 1.0000x reference)
import functools

import jax
import jax.numpy as jnp
from jax import lax
from jax.experimental import pallas as pl
from jax.experimental.pallas import tpu as pltpu
from jax.experimental.pallas import tpu_sc as plsc

_A_MUTUAL = 0.39
_N_NODES = 100000
_N_EDGES = 3200000

_NC, _NS = 2, 16
_NW = _NC * _NS
_EDGES_PER_W = _N_EDGES // _NW
_CHUNK = 2000
_NCHUNK = _EDGES_PER_W // _CHUNK
_L = 16

_N_PAD = 100352


def _table_body(pol_ref, out_ref):
    out_ref[...] = jnp.float32(_A_MUTUAL ** 0.5) * lax.rsqrt(pol_ref[...])


_TC_ROWS = 25000
_TC_BLK = 1000


def _damp_body(d_ref, g_ref, l3_ref, l5_ref):
    d = d_ref[...]
    au3 = d * d * d * g_ref[...]
    e = jnp.exp(-au3)
    l3 = 1.0 - e
    l3_ref[...] = l3
    l5_ref[...] = l3 - au3 * e


def _tc_damping(distances, g):
    spec = pl.BlockSpec((_TC_BLK, 128), lambda i: (i, 0))
    l3, l5 = pl.pallas_call(
        _damp_body,
        grid=(_TC_ROWS // _TC_BLK,),
        in_specs=[spec, spec],
        out_specs=[spec, spec],
        out_shape=[jax.ShapeDtypeStruct((_TC_ROWS, 128), jnp.float32)] * 2,
    )(distances.reshape(_TC_ROWS, 128), g.reshape(_TC_ROWS, 128))
    return l3.reshape(_N_EDGES), l5.reshape(_N_EDGES)


def _node_table(pol):
    p = jnp.pad(pol.reshape(-1), (0, _N_PAD - _N_NODES), constant_values=1.0)
    t = pl.pallas_call(
        _table_body,
        out_shape=jax.ShapeDtypeStruct((_N_PAD // 128, 128), jnp.float32),
    )(p.reshape(_N_PAD // 128, 128))
    return t.reshape(_N_PAD)


_mesh = plsc.VectorSubcoreMesh(
    core_axis_name="c", subcore_axis_name="s", num_cores=_NC, num_subcores=_NS
)


@functools.partial(
    pl.kernel,
    out_type=jax.ShapeDtypeStruct((_N_EDGES,), jnp.float32),
    mesh=_mesh,
    compiler_params=pltpu.CompilerParams(needs_layout_passes=False),
    scratch_types=[
        pltpu.VMEM((_N_PAD,), jnp.float32),
        pltpu.VMEM_SHARED((_N_PAD,), jnp.float32),
        pltpu.VMEM((_CHUNK,), jnp.int32),
        pltpu.VMEM((_CHUNK,), jnp.int32),
        pltpu.VMEM((_CHUNK,), jnp.int32),
        pltpu.VMEM((_CHUNK,), jnp.int32),
        pltpu.VMEM((_CHUNK,), jnp.float32),
        pltpu.VMEM((_CHUNK,), jnp.float32),
        pltpu.VMEM((2 * _CHUNK,), jnp.int32),
        pltpu.SemaphoreType.DMA((2,)),
        pltpu.SemaphoreType.DMA((2,)),
    ],
)
def _sc_gather_prod(t_hbm, src_hbm, dst_hbm, g_hbm,
                    table_v, table_sh, src_v0, src_v1, dst_v0, dst_v1,
                    g_v0, g_v1, dummy_in, in_sem, out_sem):
    src_b, dst_b, g_b = (src_v0, src_v1), (dst_v0, dst_v1), (g_v0, g_v1)
    wid = lax.axis_index("s") * _NC + lax.axis_index("c")
    base = wid * _EDGES_PER_W

    def start_in(j, b):
        off = base + j * _CHUNK
        pltpu.async_copy(src_hbm.at[pl.ds(off, _CHUNK)], src_b[b], in_sem.at[b])
        pltpu.async_copy(dst_hbm.at[pl.ds(off, _CHUNK)], dst_b[b], in_sem.at[b])

    def wait_in(j, b):
        pltpu.make_async_copy(src_hbm.at[pl.ds(0, 2 * _CHUNK)], dummy_in,
                              in_sem.at[b]).wait()

    def start_out(j, b):
        off = base + j * _CHUNK
        pltpu.async_copy(g_b[b], g_hbm.at[pl.ds(off, _CHUNK)], out_sem.at[b])

    def wait_out(j, b):
        off = base + j * _CHUNK
        pltpu.make_async_copy(g_b[b], g_hbm.at[pl.ds(off, _CHUNK)],
                              out_sem.at[b]).wait()

    start_in(0, 0)
    start_in(1, 1)

    @pl.when(lax.axis_index("s") == 0)
    def _():
        pltpu.sync_copy(t_hbm, table_sh)

    plsc.subcore_barrier()
    pltpu.sync_copy(table_sh, table_v)

    def compute(b):
        @plsc.parallel_loop(0, _CHUNK // _L, unroll=5)
        def _vec(i):
            s = pl.ds(i * _L, _L)
            ts = plsc.load_gather(table_v, [src_b[b][s]])
            td = plsc.load_gather(table_v, [dst_b[b][s]])
            g_b[b][s] = ts * td

    for b in range(2):
        wait_in(b, b)
        compute(b)
        start_out(b, b)
        start_in(2 + b, b)

    @pl.loop(2, _NCHUNK - 2, step=2)
    def _outer(j):
        for b in range(2):
            jj = j + b
            wait_in(jj, b)
            wait_out(jj - 2, b)
            compute(b)
            start_out(jj, b)
            start_in(jj + 2, b)

    for b in range(2):
        jj = _NCHUNK - 2 + b
        wait_in(jj, b)
        wait_out(jj - 2, b)
        compute(b)
        start_out(jj, b)

    wait_out(_NCHUNK - 2, 0)
    wait_out(_NCHUNK - 1, 1)


def kernel(species, edge_src, edge_dst, distances, vec, polarisability):
    del species, vec
    t = _node_table(polarisability)
    g = _sc_gather_prod(t, edge_src, edge_dst)
    return _tc_damping(distances, g)


# speedup vs baseline: 1.2999x; 1.2999x over previous
import functools

import jax
import jax.numpy as jnp
from jax import lax
from jax.experimental import pallas as pl
from jax.experimental.pallas import tpu as pltpu
from jax.experimental.pallas import tpu_sc as plsc

_A_MUTUAL = 0.39
_N_NODES = 100000
_N_EDGES = 3200000

_NC, _NS = 2, 16
_NW = _NC * _NS
_EDGES_PER_W = _N_EDGES // _NW
_CHUNK = 2000
_NCHUNK = _EDGES_PER_W // _CHUNK
_L = 16

_N_PAD = 100352


def _table_body(pol_ref, out_ref):
    out_ref[...] = jnp.float32(_A_MUTUAL ** 0.5) * lax.rsqrt(pol_ref[...])


def _node_table(pol):
    p = jnp.pad(pol.reshape(-1), (0, _N_PAD - _N_NODES), constant_values=1.0)
    t = pl.pallas_call(
        _table_body,
        out_shape=jax.ShapeDtypeStruct((_N_PAD // 128, 128), jnp.float32),
    )(p.reshape(_N_PAD // 128, 128))
    return t.reshape(_N_PAD)


_mesh = plsc.VectorSubcoreMesh(
    core_axis_name="c", subcore_axis_name="s", num_cores=_NC, num_subcores=_NS
)


@functools.partial(
    pl.kernel,
    out_type=(
        jax.ShapeDtypeStruct((_N_EDGES,), jnp.float32),
        jax.ShapeDtypeStruct((_N_EDGES,), jnp.float32),
    ),
    mesh=_mesh,
    compiler_params=pltpu.CompilerParams(needs_layout_passes=False),
    scratch_types=[
        pltpu.VMEM((_N_PAD,), jnp.float32),
        pltpu.VMEM_SHARED((_N_PAD,), jnp.float32),
        pltpu.VMEM((_CHUNK,), jnp.int32),
        pltpu.VMEM((_CHUNK,), jnp.int32),
        pltpu.VMEM((_CHUNK,), jnp.int32),
        pltpu.VMEM((_CHUNK,), jnp.int32),
        pltpu.VMEM((_CHUNK,), jnp.float32),
        pltpu.VMEM((_CHUNK,), jnp.float32),
        pltpu.VMEM((_CHUNK,), jnp.float32),
        pltpu.VMEM((_CHUNK,), jnp.float32),
        pltpu.VMEM((_CHUNK,), jnp.float32),
        pltpu.VMEM((_CHUNK,), jnp.float32),
        pltpu.VMEM((3 * _CHUNK,), jnp.int32),
        pltpu.VMEM((2 * _CHUNK,), jnp.float32),
        pltpu.SemaphoreType.DMA((2,)),
        pltpu.SemaphoreType.DMA((2,)),
    ],
)
def _sc_damping(t_hbm, src_hbm, dst_hbm, dist_hbm, l3_hbm, l5_hbm,
                table_v, table_sh, src_v0, src_v1, dst_v0, dst_v1, dist_v0, dist_v1,
                l3_v0, l3_v1, l5_v0, l5_v1, dummy_in, dummy_out,
                in_sem, out_sem):
    src_b, dst_b, dist_b = (src_v0, src_v1), (dst_v0, dst_v1), (dist_v0, dist_v1)
    l3_b, l5_b = (l3_v0, l3_v1), (l5_v0, l5_v1)
    wid = lax.axis_index("s") * _NC + lax.axis_index("c")
    base = wid * _EDGES_PER_W

    def start_in(j, b):
        off = base + j * _CHUNK
        pltpu.async_copy(src_hbm.at[pl.ds(off, _CHUNK)], src_b[b], in_sem.at[b])
        pltpu.async_copy(dst_hbm.at[pl.ds(off, _CHUNK)], dst_b[b], in_sem.at[b])
        pltpu.async_copy(dist_hbm.at[pl.ds(off, _CHUNK)], dist_b[b], in_sem.at[b])

    def wait_in(j, b):
        pltpu.make_async_copy(src_hbm.at[pl.ds(0, 3 * _CHUNK)], dummy_in,
                              in_sem.at[b]).wait()

    def start_out(j, b):
        off = base + j * _CHUNK
        pltpu.async_copy(l3_b[b], l3_hbm.at[pl.ds(off, _CHUNK)], out_sem.at[b])
        pltpu.async_copy(l5_b[b], l5_hbm.at[pl.ds(off, _CHUNK)], out_sem.at[b])

    def wait_out(j, b):
        pltpu.make_async_copy(dist_hbm.at[pl.ds(0, 2 * _CHUNK)], dummy_out,
                              out_sem.at[b]).wait()

    start_in(0, 0)
    start_in(1, 1)

    @pl.when(lax.axis_index("s") == 0)
    def _():
        pltpu.sync_copy(t_hbm, table_sh)

    plsc.subcore_barrier()
    pltpu.sync_copy(table_sh, table_v)

    def compute(b):
        @plsc.parallel_loop(0, _CHUNK // _L, unroll=5)
        def _vec(i):
            s = pl.ds(i * _L, _L)
            ts = plsc.load_gather(table_v, [src_b[b][s]])
            td = plsc.load_gather(table_v, [dst_b[b][s]])
            d = dist_b[b][s]
            au3 = d * d * d * ts * td
            e = jnp.exp(-au3)
            l3 = 1.0 - e
            l3_b[b][s] = l3
            l5_b[b][s] = l3 - au3 * e

    for b in range(2):
        wait_in(b, b)
        compute(b)
        start_out(b, b)
        start_in(2 + b, b)

    @pl.loop(2, _NCHUNK - 2, step=2)
    def _outer(j):
        for b in range(2):
            jj = j + b
            wait_in(jj, b)
            wait_out(jj - 2, b)
            compute(b)
            start_out(jj, b)
            start_in(jj + 2, b)

    for b in range(2):
        jj = _NCHUNK - 2 + b
        wait_in(jj, b)
        wait_out(jj - 2, b)
        compute(b)
        start_out(jj, b)

    wait_out(_NCHUNK - 2, 0)
    wait_out(_NCHUNK - 1, 1)


def kernel(species, edge_src, edge_dst, distances, vec, polarisability):
    del species, vec
    t = _node_table(polarisability)
    l3, l5 = _sc_damping(t, edge_src, edge_dst, distances)
    return (l3, l5)


# speedup vs baseline: 1.3052x; 1.0040x over previous
import functools

import jax
import jax.numpy as jnp
from jax import lax
from jax.experimental import pallas as pl
from jax.experimental.pallas import tpu as pltpu
from jax.experimental.pallas import tpu_sc as plsc

_A_MUTUAL = 0.39
_N_NODES = 100000
_N_EDGES = 3200000

_NC, _NS = 2, 16
_NW = _NC * _NS
_EDGES_PER_W = _N_EDGES // _NW
_CHUNK = 2000
_NCHUNK = _EDGES_PER_W // _CHUNK
_L = 16

_N_PAD = 100352


def _table_body(pol_ref, out_ref):
    out_ref[...] = jnp.float32(_A_MUTUAL ** 0.5) * lax.rsqrt(pol_ref[...])


def _node_table(pol):
    p = jnp.pad(pol.reshape(-1), (0, _N_PAD - _N_NODES), constant_values=1.0)
    t = pl.pallas_call(
        _table_body,
        out_shape=jax.ShapeDtypeStruct((_N_PAD // 128, 128), jnp.float32),
    )(p.reshape(_N_PAD // 128, 128))
    return t.reshape(_N_PAD)


_mesh = plsc.VectorSubcoreMesh(
    core_axis_name="c", subcore_axis_name="s", num_cores=_NC, num_subcores=_NS
)


@functools.partial(
    pl.kernel,
    out_type=(
        jax.ShapeDtypeStruct((_N_EDGES,), jnp.float32),
        jax.ShapeDtypeStruct((_N_EDGES,), jnp.float32),
    ),
    mesh=_mesh,
    compiler_params=pltpu.CompilerParams(needs_layout_passes=False),
    scratch_types=[
        pltpu.VMEM((_N_PAD,), jnp.float32),
        pltpu.VMEM_SHARED((_N_PAD,), jnp.float32),
        pltpu.VMEM((_CHUNK,), jnp.int32),
        pltpu.VMEM((_CHUNK,), jnp.int32),
        pltpu.VMEM((_CHUNK,), jnp.int32),
        pltpu.VMEM((_CHUNK,), jnp.int32),
        pltpu.VMEM((_CHUNK,), jnp.float32),
        pltpu.VMEM((_CHUNK,), jnp.float32),
        pltpu.VMEM((_CHUNK,), jnp.float32),
        pltpu.VMEM((_CHUNK,), jnp.float32),
        pltpu.VMEM((_CHUNK,), jnp.float32),
        pltpu.VMEM((_CHUNK,), jnp.float32),
        pltpu.SemaphoreType.DMA((2,)),
        pltpu.SemaphoreType.DMA((2,)),
    ],
)
def _sc_damping(t_hbm, src_hbm, dst_hbm, dist_hbm, l3_hbm, l5_hbm,
                table_v, table_sh, src_v0, src_v1, dst_v0, dst_v1, dist_v0, dist_v1,
                l3_v0, l3_v1, l5_v0, l5_v1, in_sem, out_sem):
    src_b, dst_b, dist_b = (src_v0, src_v1), (dst_v0, dst_v1), (dist_v0, dist_v1)
    l3_b, l5_b = (l3_v0, l3_v1), (l5_v0, l5_v1)
    wid = lax.axis_index("s") * _NC + lax.axis_index("c")
    base = wid * _EDGES_PER_W

    def start_in(j, b):
        off = base + j * _CHUNK
        pltpu.async_copy(src_hbm.at[pl.ds(off, _CHUNK)], src_b[b], in_sem.at[b])
        pltpu.async_copy(dst_hbm.at[pl.ds(off, _CHUNK)], dst_b[b], in_sem.at[b])
        pltpu.async_copy(dist_hbm.at[pl.ds(off, _CHUNK)], dist_b[b], in_sem.at[b])

    def wait_in(j, b):
        off = base + j * _CHUNK
        pltpu.make_async_copy(src_hbm.at[pl.ds(off, _CHUNK)], src_b[b], in_sem.at[b]).wait()
        pltpu.make_async_copy(dst_hbm.at[pl.ds(off, _CHUNK)], dst_b[b], in_sem.at[b]).wait()
        pltpu.make_async_copy(dist_hbm.at[pl.ds(off, _CHUNK)], dist_b[b], in_sem.at[b]).wait()

    def start_out(j, b):
        off = base + j * _CHUNK
        pltpu.async_copy(l3_b[b], l3_hbm.at[pl.ds(off, _CHUNK)], out_sem.at[b])
        pltpu.async_copy(l5_b[b], l5_hbm.at[pl.ds(off, _CHUNK)], out_sem.at[b])

    def wait_out(j, b):
        off = base + j * _CHUNK
        pltpu.make_async_copy(l3_b[b], l3_hbm.at[pl.ds(off, _CHUNK)], out_sem.at[b]).wait()
        pltpu.make_async_copy(l5_b[b], l5_hbm.at[pl.ds(off, _CHUNK)], out_sem.at[b]).wait()

    start_in(0, 0)
    start_in(1, 1)

    @pl.when(lax.axis_index("s") == 0)
    def _():
        pltpu.sync_copy(t_hbm, table_sh)

    plsc.subcore_barrier()
    pltpu.sync_copy(table_sh, table_v)

    @pl.loop(0, _NCHUNK, step=2)
    def _outer(j):
        for b in range(2):
            jj = j + b
            wait_in(jj, b)

            @pl.when(jj >= 2)
            def _():
                wait_out(jj - 2, b)

            @plsc.parallel_loop(0, _CHUNK // _L, unroll=5)
            def _vec(i):
                s = pl.ds(i * _L, _L)
                ts = plsc.load_gather(table_v, [src_b[b][s]])
                td = plsc.load_gather(table_v, [dst_b[b][s]])
                d = dist_b[b][s]
                au3 = d * d * d * ts * td
                e = jnp.exp(-au3)
                l3 = 1.0 - e
                l3_b[b][s] = l3
                l5_b[b][s] = l3 - au3 * e

            start_out(jj, b)

            @pl.when(jj + 2 < _NCHUNK)
            def _():
                start_in(jj + 2, b)

    wait_out(_NCHUNK - 2, 0)
    wait_out(_NCHUNK - 1, 1)


def kernel(species, edge_src, edge_dst, distances, vec, polarisability):
    del species, vec
    t = _node_table(polarisability)
    l3, l5 = _sc_damping(t, edge_src, edge_dst, distances)
    return (l3, l5)
